# baseline (speedup 1.0000x reference)
import functools
import jax
import jax.numpy as jnp
from jax import lax
from jax.experimental import pallas as pl
from jax.experimental.pallas import tpu as pltpu
from jax.experimental.pallas import tpu_sc as plsc

N = 100000
E = 1600000
D_IN = 128
D_HID = 50
D_OUT = 7

R = 100352
NCH = 4
DK = 16
DK2 = 8
K = 5
NSTEPS = 80
EPAD = 2 * 16 * NSTEPS * K * 128
RB = 1000
RPT = R // 16
ZR = 128
NZ = RPT // ZR


def _sc_mesh():
    return plsc.VectorSubcoreMesh(core_axis_name="c", subcore_axis_name="s")


def _fill(ref, nrows, vec):
    def body(i, carry):
        ref[i, :] = vec
        return carry
    lax.fori_loop(0, nrows, body, 0)


def _sc_deg(sd):
    @functools.partial(
        pl.kernel,
        out_type=jax.ShapeDtypeStruct((2, R, DK2), jnp.float32),
        mesh=_sc_mesh(),
        compiler_params=pltpu.CompilerParams(use_tc_tiling_on_sc=False),
        scratch_types=[
            pltpu.VMEM((K, 128), jnp.int32),
            pltpu.VMEM((K, 128), jnp.int32),
            pltpu.VMEM((128, DK2), jnp.float32),
            pltpu.VMEM((ZR, DK2), jnp.float32),
            pltpu.VMEM_SHARED((R, DK2), jnp.float32),
            pltpu.SemaphoreType.DMA,
            pltpu.SemaphoreType.DMA,
            pltpu.SemaphoreType.DMA,
            pltpu.SemaphoreType.DMA,
        ],
    )
    def k(sd_hbm, out_hbm, didxA, didxB, onesv, obuf, acc,
          sem_iA, sem_iB, sem_sA, sem_sB):
        didx = (didxA, didxB)
        sem_i = (sem_iA, sem_iB)
        sem_s = (sem_sA, sem_sB)
        c = lax.axis_index("c")
        s = lax.axis_index("s")
        _fill(onesv, 128, jnp.ones((DK2,), jnp.float32))
        _fill(obuf, ZR, jnp.zeros((DK2,), jnp.float32))
        for z in range(NZ):
            pltpu.sync_copy(obuf, acc.at[pl.ds(s * RPT + z * ZR, ZR)])
        plsc.subcore_barrier()

        def load_idx(i, r):
            return pltpu.async_copy(sd_hbm.at[c, s, i, 1], didx[r], sem_i[r])

        def fire_scatter(r):
            for j in range(K):
                pltpu.async_copy(
                    onesv, acc.at[didx[r].at[j]], sem_s[r], add=True)

        def wait_scatter(r):
            for j in range(K):
                pltpu.make_async_copy(
                    onesv, acc.at[didx[r].at[j]], sem_s[r]).wait()

        load_idx(0, 0).wait()
        fire_scatter(0)
        load_idx(1, 1)

        def pair(io, carry):
            for b in range(2):
                i = 2 * io + b
                r0, r1 = b, 1 - b
                pltpu.make_async_copy(
                    sd_hbm.at[c, s, i + 1, 1], didx[r1], sem_i[r1]).wait()
                fire_scatter(r1)
                wait_scatter(r0)
                load_idx(i + 2, r0)
            return carry

        lax.fori_loop(0, (NSTEPS - 2) // 2, pair, 0)
        pltpu.make_async_copy(
            sd_hbm.at[c, s, NSTEPS - 1, 1], didx[1], sem_i[1]).wait()
        fire_scatter(1)
        wait_scatter(0)
        wait_scatter(1)

        plsc.subcore_barrier()
        for z in range(NZ):
            pltpu.sync_copy(acc.at[pl.ds(s * RPT + z * ZR, ZR)], obuf)
            pltpu.sync_copy(obuf, out_hbm.at[c, pl.ds(s * RPT + z * ZR, ZR)])

    return k(sd)


def _sc_agg(nch, w):
    @functools.partial(
        pl.kernel,
        out_type=jax.ShapeDtypeStruct((nch, 2, R, w), jnp.float32),
        mesh=_sc_mesh(),
        compiler_params=pltpu.CompilerParams(use_tc_tiling_on_sc=False),
        scratch_types=[
            pltpu.VMEM((2, K, 128), jnp.int32),
            pltpu.VMEM((2, K, 128), jnp.int32),
            pltpu.VMEM((K, 128, w), jnp.float32),
            pltpu.VMEM((K, 128, w), jnp.float32),
            pltpu.VMEM((ZR, w), jnp.float32),
            pltpu.VMEM((ZR, w), jnp.float32),
            pltpu.VMEM_SHARED((R, w), jnp.float32),
            pltpu.SemaphoreType.DMA,
            pltpu.SemaphoreType.DMA,
            pltpu.SemaphoreType.DMA,
            pltpu.SemaphoreType.DMA,
            pltpu.SemaphoreType.DMA,
            pltpu.SemaphoreType.DMA,
        ],
    )
    def k(sd_hbm, *rest):
        tabs = rest[:nch]
        out_hbm = rest[nch]
        (ibufA, ibufB, gbufA, gbufB, zbuf, obuf, acc,
         sem_iA, sem_iB, sem_gA, sem_gB, sem_sA, sem_sB) = rest[nch + 1:]
        ibuf = (ibufA, ibufB)
        gbuf = (gbufA, gbufB)
        sem_i = (sem_iA, sem_iB)
        sem_g = (sem_gA, sem_gB)
        sem_s = (sem_sA, sem_sB)
        c = lax.axis_index("c")
        s = lax.axis_index("s")
        _fill(zbuf, ZR, jnp.zeros((w,), jnp.float32))

        for ch in range(nch):
            tab = tabs[ch]

            def load_idx(i, r):
                return pltpu.async_copy(sd_hbm.at[c, s, i], ibuf[r], sem_i[r])

            def fire(r):
                for j in range(K):
                    pltpu.async_copy(
                        tab.at[ibuf[r].at[0, j]], gbuf[r].at[j], sem_g[r])

            def drain_scatter(r):
                for j in range(K):
                    pltpu.make_async_copy(
                        tab.at[ibuf[r].at[0, j]], gbuf[r].at[j],
                        sem_g[r]).wait()
                    pltpu.async_copy(
                        gbuf[r].at[j], acc.at[ibuf[r].at[1, j]], sem_s[r],
                        add=True)
                for j in range(K):
                    pltpu.make_async_copy(
                        gbuf[r].at[j], acc.at[ibuf[r].at[1, j]],
                        sem_s[r]).wait()

            for z in range(NZ):
                pltpu.sync_copy(zbuf, acc.at[pl.ds(s * RPT + z * ZR, ZR)])
            plsc.subcore_barrier()

            load_idx(0, 0).wait()
            fire(0)
            load_idx(1, 1)

            def pair(io, carry):
                for b in range(2):
                    i = 2 * io + b
                    r0, r1 = b, 1 - b
                    pltpu.make_async_copy(
                        sd_hbm.at[c, s, i + 1], ibuf[r1], sem_i[r1]).wait()
                    fire(r1)
                    drain_scatter(r0)
                    load_idx(i + 2, r0)
                return carry

            lax.fori_loop(0, (NSTEPS - 2) // 2, pair, 0)
            pltpu.make_async_copy(
                sd_hbm.at[c, s, NSTEPS - 1], ibuf[1], sem_i[1]).wait()
            fire(1)
            drain_scatter(0)
            drain_scatter(1)

            plsc.subcore_barrier()
            for z in range(NZ):
                pltpu.sync_copy(acc.at[pl.ds(s * RPT + z * ZR, ZR)], obuf)
                pltpu.sync_copy(
                    obuf, out_hbm.at[ch, c, pl.ds(s * RPT + z * ZR, ZR)])

    return k


def _tc_b1_body(xb, w1, p_o):
    p_o[...] = jnp.dot(xb[...], w1[...], preferred_element_type=jnp.float32)


def _tc_b1(x, w1p):
    return pl.pallas_call(
        _tc_b1_body,
        grid=(N // RB,),
        in_specs=[
            pl.BlockSpec((RB, D_IN), lambda i: (i, 0)),
            pl.BlockSpec((D_IN, 64), lambda i: (0, 0)),
        ],
        out_specs=pl.BlockSpec((RB, 64), lambda i: (i, 0)),
        out_shape=jax.ShapeDtypeStruct((N, 64), jnp.float32),
    )(x, w1p)


def _tc_b2_body(deg_p, p, dinv_o, g0, g1, g2, g3):
    deg = deg_p[0, :, 0:1] + deg_p[1, :, 0:1] + 1.0
    dinv = lax.rsqrt(deg)
    dinv_o[...] = jnp.broadcast_to(dinv, dinv_o.shape)
    g = p[...] * dinv
    g0[...] = g[:, 0:16]
    g1[...] = g[:, 16:32]
    g2[...] = g[:, 32:48]
    g3[...] = g[:, 48:64]


def _tc_b2(deg_p, p):
    return pl.pallas_call(
        _tc_b2_body,
        grid=(N // RB,),
        in_specs=[
            pl.BlockSpec((2, RB, DK2), lambda i: (0, i, 0)),
            pl.BlockSpec((RB, 64), lambda i: (i, 0)),
        ],
        out_specs=[
            pl.BlockSpec((RB, 16), lambda i: (i, 0)),
            pl.BlockSpec((RB, DK), lambda i: (i, 0)),
            pl.BlockSpec((RB, DK), lambda i: (i, 0)),
            pl.BlockSpec((RB, DK), lambda i: (i, 0)),
            pl.BlockSpec((RB, DK), lambda i: (i, 0)),
        ],
        out_shape=[
            jax.ShapeDtypeStruct((N, 16), jnp.float32),
            jax.ShapeDtypeStruct((N, DK), jnp.float32),
            jax.ShapeDtypeStruct((N, DK), jnp.float32),
            jax.ShapeDtypeStruct((N, DK), jnp.float32),
            jax.ShapeDtypeStruct((N, DK), jnp.float32),
        ],
    )(deg_p, p)


def _tc_d_body(agg_p, g0, g1, g2, g3, dinvb, b1, w2, g2_o):
    dinv = dinvb[:, 0:1]
    parts = []
    for c in range(NCH):
        parts.append(agg_p[c, 0] + agg_p[c, 1])
    a = jnp.concatenate(parts, axis=1)
    gself = jnp.concatenate([g0[...], g1[...], g2[...], g3[...]], axis=1)
    h1 = jnp.maximum(dinv * (a + gself) + b1[...], 0.0)
    g2v = jnp.dot(h1, w2[...], preferred_element_type=jnp.float32)
    g2_o[...] = g2v * dinv


def _tc_d(agg1_p, g1c, dinvb, b1p, w2p):
    return pl.pallas_call(
        _tc_d_body,
        grid=(N // RB,),
        in_specs=[
            pl.BlockSpec((NCH, 2, RB, DK), lambda i: (0, 0, i, 0)),
            pl.BlockSpec((RB, DK), lambda i: (i, 0)),
            pl.BlockSpec((RB, DK), lambda i: (i, 0)),
            pl.BlockSpec((RB, DK), lambda i: (i, 0)),
            pl.BlockSpec((RB, DK), lambda i: (i, 0)),
            pl.BlockSpec((RB, 16), lambda i: (i, 0)),
            pl.BlockSpec((1, 64), lambda i: (0, 0)),
            pl.BlockSpec((64, DK2), lambda i: (0, 0)),
        ],
        out_specs=pl.BlockSpec((RB, DK2), lambda i: (i, 0)),
        out_shape=jax.ShapeDtypeStruct((N, DK2), jnp.float32),
    )(agg1_p, *g1c, dinvb, b1p, w2p)


def _tc_f_body(agg_p, g2, dinvb, b2, out_o):
    dinv = dinvb[:, 0:1]
    a = agg_p[0] + agg_p[1] + g2[...]
    z = dinv * a + b2[...]
    out_o[...] = 1.0 / (1.0 + jnp.exp(-z[:, 0:D_OUT]))


def _tc_f(agg2_p, g2, dinvb, b2p):
    return pl.pallas_call(
        _tc_f_body,
        grid=(N // RB,),
        in_specs=[
            pl.BlockSpec((2, RB, DK2), lambda i: (0, i, 0)),
            pl.BlockSpec((RB, DK2), lambda i: (i, 0)),
            pl.BlockSpec((RB, 16), lambda i: (i, 0)),
            pl.BlockSpec((1, DK2), lambda i: (0, 0)),
        ],
        out_specs=pl.BlockSpec((RB, D_OUT), lambda i: (i, 0)),
        out_shape=jax.ShapeDtypeStruct((N, D_OUT), jnp.float32),
    )(agg2_p, g2, dinvb, b2p)


def kernel(x, edge_index, W1, b1, W2, b2):
    src = edge_index[0]
    dst = edge_index[1]
    pad = EPAD - E
    pad_dst = (N + (jnp.arange(pad, dtype=jnp.int32) % 256)).astype(jnp.int32)
    srcp = jnp.concatenate([src, jnp.zeros((pad,), jnp.int32)])
    dstp = jnp.concatenate([dst, pad_dst])
    sd = jnp.stack([srcp.reshape(2, 16, NSTEPS, K, 128),
                    dstp.reshape(2, 16, NSTEPS, K, 128)], axis=3)

    W1p = jnp.pad(W1, ((0, 0), (0, 64 - D_HID)))
    b1p = jnp.pad(b1, (0, 64 - D_HID)).reshape(1, 64)
    W2p = jnp.pad(W2, ((0, 64 - D_HID), (0, DK2 - D_OUT)))
    b2p = jnp.pad(b2, (0, DK2 - D_OUT)).reshape(1, DK2)

    p = _tc_b1(x, W1p)
    deg_p = _sc_deg(sd)
    dinvb, *g1c = _tc_b2(deg_p, p)

    agg1_p = _sc_agg(NCH, DK)(sd, *g1c)
    g2 = _tc_d(agg1_p, g1c, dinvb, b1p, W2p)

    agg2_p = _sc_agg(1, DK2)(sd, g2)[0]
    return _tc_f(agg2_p, g2, dinvb, b2p)


# speedup vs baseline: 1.5045x; 1.5045x over previous
import functools
import jax
import jax.numpy as jnp
from jax import lax
from jax.experimental import pallas as pl
from jax.experimental.pallas import tpu as pltpu
from jax.experimental.pallas import tpu_sc as plsc

N = 100000
E = 1600000
D_IN = 128
D_HID = 50
D_OUT = 7

R = 102400
NCH = 4
DK = 16
DK2 = 8
K = 4
NSTEPS = 98
EPAD = 2 * 16 * NSTEPS * K * 128
RB = 1000
RPT = R // 16
ZR = 128
NZ = RPT // ZR


def _sc_mesh():
    return plsc.VectorSubcoreMesh(core_axis_name="c", subcore_axis_name="s")


def _fill(ref, nrows, vec):
    def body(i, carry):
        ref[i, :] = vec
        return carry
    lax.fori_loop(0, nrows, body, 0)


def _sc_deg(sd):
    @functools.partial(
        pl.kernel,
        out_type=jax.ShapeDtypeStruct((2, R, DK2), jnp.float32),
        mesh=_sc_mesh(),
        compiler_params=pltpu.CompilerParams(use_tc_tiling_on_sc=False),
        scratch_types=[
            pltpu.VMEM((K, 128), jnp.int32),
            pltpu.VMEM((K, 128), jnp.int32),
            pltpu.VMEM((128, DK2), jnp.float32),
            pltpu.VMEM((ZR, DK2), jnp.float32),
            pltpu.VMEM_SHARED((R, DK2), jnp.float32),
            pltpu.SemaphoreType.DMA,
            pltpu.SemaphoreType.DMA,
            pltpu.SemaphoreType.DMA,
            pltpu.SemaphoreType.DMA,
        ],
    )
    def k(sd_hbm, out_hbm, didxA, didxB, onesv, obuf, acc,
          sem_iA, sem_iB, sem_sA, sem_sB):
        didx = (didxA, didxB)
        sem_i = (sem_iA, sem_iB)
        sem_s = (sem_sA, sem_sB)
        c = lax.axis_index("c")
        s = lax.axis_index("s")
        _fill(onesv, 128, jnp.ones((DK2,), jnp.float32))
        _fill(obuf, ZR, jnp.zeros((DK2,), jnp.float32))
        for z in range(NZ):
            pltpu.sync_copy(obuf, acc.at[pl.ds(s * RPT + z * ZR, ZR)])
        plsc.subcore_barrier()

        def load_idx(i, r):
            return pltpu.async_copy(sd_hbm.at[c, s, i, 1], didx[r], sem_i[r])

        def fire_scatter(r):
            for j in range(K):
                pltpu.async_copy(
                    onesv, acc.at[didx[r].at[j]], sem_s[r], add=True)

        def wait_scatter(r):
            for j in range(K):
                pltpu.make_async_copy(
                    onesv, acc.at[didx[r].at[j]], sem_s[r]).wait()

        load_idx(0, 0).wait()
        fire_scatter(0)
        load_idx(1, 1)

        def pair(io, carry):
            for b in range(2):
                i = 2 * io + b
                r0, r1 = b, 1 - b
                pltpu.make_async_copy(
                    sd_hbm.at[c, s, i + 1, 1], didx[r1], sem_i[r1]).wait()
                fire_scatter(r1)
                wait_scatter(r0)
                load_idx(i + 2, r0)
            return carry

        lax.fori_loop(0, (NSTEPS - 2) // 2, pair, 0)
        pltpu.make_async_copy(
            sd_hbm.at[c, s, NSTEPS - 1, 1], didx[1], sem_i[1]).wait()
        fire_scatter(1)
        wait_scatter(0)
        wait_scatter(1)

        plsc.subcore_barrier()
        for z in range(NZ):
            pltpu.sync_copy(acc.at[pl.ds(s * RPT + z * ZR, ZR)], obuf)
            pltpu.sync_copy(obuf, out_hbm.at[c, pl.ds(s * RPT + z * ZR, ZR)])

    return k(sd)


def _sc_agg(nch, w):
    @functools.partial(
        pl.kernel,
        out_type=jax.ShapeDtypeStruct((nch, 2, R, w), jnp.float32),
        mesh=_sc_mesh(),
        compiler_params=pltpu.CompilerParams(use_tc_tiling_on_sc=False),
        scratch_types=[
            pltpu.VMEM((2, K, 128), jnp.int32),
            pltpu.VMEM((2, K, 128), jnp.int32),
            pltpu.VMEM((K, 128, w), jnp.float32),
            pltpu.VMEM((K, 128, w), jnp.float32),
            pltpu.VMEM((ZR, w), jnp.float32),
            pltpu.VMEM((ZR, w), jnp.float32),
            pltpu.VMEM_SHARED((R, w), jnp.float32),
            pltpu.SemaphoreType.DMA,
            pltpu.SemaphoreType.DMA,
            pltpu.SemaphoreType.DMA,
            pltpu.SemaphoreType.DMA,
            pltpu.SemaphoreType.DMA,
            pltpu.SemaphoreType.DMA,
        ],
    )
    def k(sd_hbm, *rest):
        tabs = rest[:nch]
        out_hbm = rest[nch]
        (ibufA, ibufB, gbufA, gbufB, zbuf, obuf, acc,
         sem_iA, sem_iB, sem_gA, sem_gB, sem_sA, sem_sB) = rest[nch + 1:]
        ibuf = (ibufA, ibufB)
        gbuf = (gbufA, gbufB)
        sem_i = (sem_iA, sem_iB)
        sem_g = (sem_gA, sem_gB)
        sem_s = (sem_sA, sem_sB)
        c = lax.axis_index("c")
        s = lax.axis_index("s")
        _fill(zbuf, ZR, jnp.zeros((w,), jnp.float32))

        for ch in range(nch):
            tab = tabs[ch]

            def load_idx(i, r):
                return pltpu.async_copy(sd_hbm.at[c, s, i], ibuf[r], sem_i[r])

            def fire(r):
                for j in range(K):
                    pltpu.async_copy(
                        tab.at[ibuf[r].at[0, j]], gbuf[r].at[j], sem_g[r])

            def drain_scatter(r):
                for j in range(K):
                    pltpu.make_async_copy(
                        tab.at[ibuf[r].at[0, j]], gbuf[r].at[j],
                        sem_g[r]).wait()
                    pltpu.async_copy(
                        gbuf[r].at[j], acc.at[ibuf[r].at[1, j]], sem_s[r],
                        add=True)
                for j in range(K):
                    pltpu.make_async_copy(
                        gbuf[r].at[j], acc.at[ibuf[r].at[1, j]],
                        sem_s[r]).wait()

            for z in range(NZ):
                pltpu.sync_copy(zbuf, acc.at[pl.ds(s * RPT + z * ZR, ZR)])
            plsc.subcore_barrier()

            load_idx(0, 0).wait()
            fire(0)
            load_idx(1, 1)

            def pair(io, carry):
                for b in range(2):
                    i = 2 * io + b
                    r0, r1 = b, 1 - b
                    pltpu.make_async_copy(
                        sd_hbm.at[c, s, i + 1], ibuf[r1], sem_i[r1]).wait()
                    fire(r1)
                    drain_scatter(r0)
                    load_idx(i + 2, r0)
                return carry

            lax.fori_loop(0, (NSTEPS - 2) // 2, pair, 0)
            pltpu.make_async_copy(
                sd_hbm.at[c, s, NSTEPS - 1], ibuf[1], sem_i[1]).wait()
            fire(1)
            drain_scatter(0)
            drain_scatter(1)

            plsc.subcore_barrier()
            for z in range(NZ):
                pltpu.sync_copy(acc.at[pl.ds(s * RPT + z * ZR, ZR)], obuf)
                pltpu.sync_copy(
                    obuf, out_hbm.at[ch, c, pl.ds(s * RPT + z * ZR, ZR)])

    return k


def _tc_b1_body(xb, w1, p_o):
    p_o[...] = jnp.dot(xb[...], w1[...], preferred_element_type=jnp.float32)


def _tc_b1(x, w1p):
    return pl.pallas_call(
        _tc_b1_body,
        grid=(N // RB,),
        in_specs=[
            pl.BlockSpec((RB, D_IN), lambda i: (i, 0)),
            pl.BlockSpec((D_IN, 64), lambda i: (0, 0)),
        ],
        out_specs=pl.BlockSpec((RB, 64), lambda i: (i, 0)),
        out_shape=jax.ShapeDtypeStruct((N, 64), jnp.float32),
    )(x, w1p)


def _tc_b2_body(deg_p, p, dinv_o, g0, g1, g2, g3):
    deg = deg_p[0, :, 0:1] + deg_p[1, :, 0:1] + 1.0
    dinv = lax.rsqrt(deg)
    dinv_o[...] = jnp.broadcast_to(dinv, dinv_o.shape)
    g = p[...] * dinv
    g0[...] = g[:, 0:16]
    g1[...] = g[:, 16:32]
    g2[...] = g[:, 32:48]
    g3[...] = g[:, 48:64]


def _tc_b2(deg_p, p):
    return pl.pallas_call(
        _tc_b2_body,
        grid=(N // RB,),
        in_specs=[
            pl.BlockSpec((2, RB, DK2), lambda i: (0, i, 0)),
            pl.BlockSpec((RB, 64), lambda i: (i, 0)),
        ],
        out_specs=[
            pl.BlockSpec((RB, 16), lambda i: (i, 0)),
            pl.BlockSpec((RB, DK), lambda i: (i, 0)),
            pl.BlockSpec((RB, DK), lambda i: (i, 0)),
            pl.BlockSpec((RB, DK), lambda i: (i, 0)),
            pl.BlockSpec((RB, DK), lambda i: (i, 0)),
        ],
        out_shape=[
            jax.ShapeDtypeStruct((N, 16), jnp.float32),
            jax.ShapeDtypeStruct((N, DK), jnp.float32),
            jax.ShapeDtypeStruct((N, DK), jnp.float32),
            jax.ShapeDtypeStruct((N, DK), jnp.float32),
            jax.ShapeDtypeStruct((N, DK), jnp.float32),
        ],
    )(deg_p, p)


def _tc_d_body(agg_p, g0, g1, g2, g3, dinvb, b1, w2, g2_o):
    dinv = dinvb[:, 0:1]
    parts = []
    for c in range(NCH):
        parts.append(agg_p[c, 0] + agg_p[c, 1])
    a = jnp.concatenate(parts, axis=1)
    gself = jnp.concatenate([g0[...], g1[...], g2[...], g3[...]], axis=1)
    h1 = jnp.maximum(dinv * (a + gself) + b1[...], 0.0)
    g2v = jnp.dot(h1, w2[...], preferred_element_type=jnp.float32)
    g2_o[...] = g2v * dinv


def _tc_d(agg1_p, g1c, dinvb, b1p, w2p):
    return pl.pallas_call(
        _tc_d_body,
        grid=(N // RB,),
        in_specs=[
            pl.BlockSpec((NCH, 2, RB, DK), lambda i: (0, 0, i, 0)),
            pl.BlockSpec((RB, DK), lambda i: (i, 0)),
            pl.BlockSpec((RB, DK), lambda i: (i, 0)),
            pl.BlockSpec((RB, DK), lambda i: (i, 0)),
            pl.BlockSpec((RB, DK), lambda i: (i, 0)),
            pl.BlockSpec((RB, 16), lambda i: (i, 0)),
            pl.BlockSpec((1, 64), lambda i: (0, 0)),
            pl.BlockSpec((64, DK2), lambda i: (0, 0)),
        ],
        out_specs=pl.BlockSpec((RB, DK2), lambda i: (i, 0)),
        out_shape=jax.ShapeDtypeStruct((N, DK2), jnp.float32),
    )(agg1_p, *g1c, dinvb, b1p, w2p)


def _tc_f_body(agg_p, g2, dinvb, b2, out_o):
    dinv = dinvb[:, 0:1]
    a = agg_p[0] + agg_p[1] + g2[...]
    z = dinv * a + b2[...]
    out_o[...] = 1.0 / (1.0 + jnp.exp(-z[:, 0:D_OUT]))


def _tc_f(agg2_p, g2, dinvb, b2p):
    return pl.pallas_call(
        _tc_f_body,
        grid=(N // RB,),
        in_specs=[
            pl.BlockSpec((2, RB, DK2), lambda i: (0, i, 0)),
            pl.BlockSpec((RB, DK2), lambda i: (i, 0)),
            pl.BlockSpec((RB, 16), lambda i: (i, 0)),
            pl.BlockSpec((1, DK2), lambda i: (0, 0)),
        ],
        out_specs=pl.BlockSpec((RB, D_OUT), lambda i: (i, 0)),
        out_shape=jax.ShapeDtypeStruct((N, D_OUT), jnp.float32),
    )(agg2_p, g2, dinvb, b2p)


def kernel(x, edge_index, W1, b1, W2, b2):
    src = edge_index[0]
    dst = edge_index[1]
    pad = EPAD - E
    pad_dst = (N + (jnp.arange(pad, dtype=jnp.int32) % 2048)).astype(jnp.int32)
    srcp = jnp.concatenate([src, jnp.arange(pad, dtype=jnp.int32) % 8192])
    dstp = jnp.concatenate([dst, pad_dst])
    sd = jnp.stack([srcp.reshape(2, 16, NSTEPS, K, 128),
                    dstp.reshape(2, 16, NSTEPS, K, 128)], axis=3)

    W1p = jnp.pad(W1, ((0, 0), (0, 64 - D_HID)))
    b1p = jnp.pad(b1, (0, 64 - D_HID)).reshape(1, 64)
    W2p = jnp.pad(W2, ((0, 64 - D_HID), (0, DK2 - D_OUT)))
    b2p = jnp.pad(b2, (0, DK2 - D_OUT)).reshape(1, DK2)

    p = _tc_b1(x, W1p)
    deg_p = _sc_deg(sd)
    dinvb, *g1c = _tc_b2(deg_p, p)

    agg1_p = _sc_agg(NCH, DK)(sd, *g1c)
    g2 = _tc_d(agg1_p, g1c, dinvb, b1p, W2p)

    agg2_p = _sc_agg(1, DK2)(sd, g2)[0]
    return _tc_f(agg2_p, g2, dinvb, b2p)
